# Initial kernel scaffold; baseline (speedup 1.0000x reference)
import jax
import jax.numpy as jnp
from jax.experimental import pallas as pl


def kernel(hidden_states, other_seq_emb, num_docs):
    raise NotImplementedError("write your pallas kernel here")


# speedup vs baseline: 1.1537x; 1.1537x over previous
import jax
import jax.numpy as jnp
from jax.experimental import pallas as pl

_BD = 8


def _concat_kernel(hs_ref, cls_ref, emb_ref, out_ref):
    seq_len = hs_ref.shape[1]
    out_ref[:, :seq_len, :] = hs_ref[...]
    tail = cls_ref[:, 1, :] + emb_ref[0]
    out_ref[:, seq_len:, :] = jnp.broadcast_to(
        tail[None], (out_ref.shape[0],) + tail.shape
    )


def kernel(hidden_states, other_seq_emb, num_docs):
    total_docs, seq_len, hidden = hidden_states.shape
    n_queries = num_docs.shape[0]
    depth = total_docs // n_queries
    bd = _BD
    grid = (total_docs // bd,)
    blocks_per_query = depth // bd
    out = pl.pallas_call(
        _concat_kernel,
        grid=grid,
        in_specs=[
            pl.BlockSpec((bd, seq_len, hidden), lambda i: (i, 0, 0)),
            pl.BlockSpec((depth, 8, hidden),
                         lambda i: (i // blocks_per_query, 0, 0)),
            pl.BlockSpec((1, hidden), lambda i: (0, 0)),
        ],
        out_specs=pl.BlockSpec((bd, seq_len + depth, hidden),
                               lambda i: (i, 0, 0)),
        out_shape=jax.ShapeDtypeStruct(
            (total_docs, seq_len + depth, hidden), hidden_states.dtype),
    )(hidden_states, hidden_states, other_seq_emb)
    return out
